# baseline (speedup 1.0000x reference)
import math

import numpy as np
import jax
import jax.numpy as jnp
from jax.experimental import pallas as pl
from jax.experimental.pallas import tpu as pltpu

B = 2
P = 2048
D = 1024
H = 16
DK = 64
FF = 32
HOR = 24
NL = 2
L = D
U = 6
SCALE = float(DK ** 0.5)


def _pe_np():
    pe = np.zeros((L, D), np.float32)
    pos = np.arange(L, dtype=np.float32)[:, None]
    div = np.exp(np.arange(0, D, 2, dtype=np.float32) * (-math.log(10000.0) / D))
    pe[:, 0::2] = np.sin(pos * div)
    pe[:, 1::2] = np.cos(pos * div)
    return pe


_PE = _pe_np()

_RB = 256


def _embed_body(x_ref, w_ref, b_ref, pe_ref, o_ref):
    xb = x_ref[0]
    wb = w_ref[...]
    acc = jax.lax.dot_general(wb, xb, (((0,), (0,)), ((), ())),
                              preferred_element_type=jnp.float32)
    o_ref[0] = acc + b_ref[...] + pe_ref[...]


def _embed(x, emb_w, emb_b):
    return pl.pallas_call(
        _embed_body,
        grid=(B, L // _RB),
        in_specs=[
            pl.BlockSpec((1, P, D), lambda b, j: (b, 0, 0)),
            pl.BlockSpec((P, _RB), lambda b, j: (0, j)),
            pl.BlockSpec((_RB, 1), lambda b, j: (j, 0)),
            pl.BlockSpec((_RB, D), lambda b, j: (j, 0)),
        ],
        out_specs=pl.BlockSpec((1, _RB, D), lambda b, j: (b, j, 0)),
        out_shape=jax.ShapeDtypeStruct((B, L, D), jnp.float32),
    )(x, emb_w, emb_b.reshape(D, 1), jnp.asarray(_PE))


def _net_body(h0_ref, qw_ref, kw_ref, vw_ref, ow_ref, qb_ref, kb_ref, vb_ref,
              ob_ref, g1_ref, b1_ref, w1_ref, fb1_ref, w2_ref, fb2_ref,
              g2_ref, b2_ref, fcw_ref, fcb_ref, o_ref, hs_ref):
    i = pl.program_id(1)

    @pl.when(i == 0)
    def _():
        hs_ref[...] = h0_ref[0]

    hb = hs_ref[...]
    hb16 = hb.astype(jnp.bfloat16)
    q_all = jnp.dot(hb16, qw_ref[0],
                    preferred_element_type=jnp.float32) + qb_ref[0]
    k_all = jnp.dot(hb16, kw_ref[0],
                    preferred_element_type=jnp.float32) + kb_ref[0]
    q16 = q_all.astype(jnp.bfloat16)
    k16 = k_all.astype(jnp.bfloat16)
    io_row = jax.lax.broadcasted_iota(jnp.int32, (1, L), 1)
    m_list = []
    for hh in range(H):
        sl = slice(hh * DK, (hh + 1) * DK)
        st = jax.lax.dot_general(k16[:, sl], q16[:, sl],
                                 (((1,), (1,)), ((), ())),
                                 preferred_element_type=jnp.float32)
        m_list.append(jnp.max(st, axis=0, keepdims=True))
    m2 = jnp.concatenate(m_list, axis=0)
    idx_cols = []
    for _ in range(U):
        vmax = jnp.max(m2, axis=1, keepdims=True)
        idx2 = jnp.min(jnp.where(m2 == vmax, io_row, jnp.int32(L)),
                       axis=1, keepdims=True)
        idx_cols.append(idx2)
        m2 = jnp.where(io_row == idx2, jnp.float32(-jnp.inf), m2)
    pad = jnp.full((H, 1), jnp.int32(L), jnp.int32)
    idx_mat = jnp.concatenate(idx_cols + [pad, pad], axis=1)
    e_all = (idx_mat[:, :, None] == io_row.reshape(1, 1, L)
             ).astype(jnp.float32).reshape(H * 8, L)
    rg = jax.lax.broadcasted_iota(jnp.int32, (H * 8, 1), 0) // 8
    maskc = (rg == io_row // DK).astype(jnp.float32)
    q_sel = jnp.dot(e_all, q_all, preferred_element_type=jnp.float32) * maskc
    a = jax.lax.dot_general(q_sel, k_all, (((1,), (1,)), ((), ())),
                            preferred_element_type=jnp.float32) / SCALE
    a = a - jnp.max(a, axis=1, keepdims=True)
    w = jnp.exp(a)
    w = w / jnp.sum(w, axis=1, keepdims=True)
    wh = jnp.dot(w.astype(jnp.bfloat16), hb16,
                 preferred_element_type=jnp.float32)
    ctx = (jnp.dot(wh.astype(jnp.bfloat16), vw_ref[0],
                   preferred_element_type=jnp.float32)
           + vb_ref[0]) * maskc
    cf = jnp.dot(ctx.astype(jnp.bfloat16), ow_ref[0],
                 preferred_element_type=jnp.float32)
    delta = jax.lax.dot_general(e_all, cf, (((0,), (0,)), ((), ())),
                                preferred_element_type=jnp.float32)
    y = hb + delta + ob_ref[0]
    mu = jnp.mean(y, axis=1, keepdims=True)
    var = jnp.mean(y * y, axis=1, keepdims=True) - mu * mu
    hn = (y - mu) / jnp.sqrt(var + 1e-5) * g1_ref[0] + b1_ref[0]
    f = jnp.maximum(
        jnp.dot(hn, w1_ref[0], preferred_element_type=jnp.float32) + fb1_ref[0],
        0.0)
    f = jnp.dot(f, w2_ref[0], preferred_element_type=jnp.float32) + fb2_ref[0]
    z = hn + f
    mu2 = jnp.mean(z, axis=1, keepdims=True)
    var2 = jnp.mean(z * z, axis=1, keepdims=True) - mu2 * mu2
    zf = (z - mu2) / jnp.sqrt(var2 + 1e-5) * g2_ref[0] + b2_ref[0]
    hs_ref[...] = zf
    pooled = jnp.mean(zf, axis=0, keepdims=True)
    o_ref[0] = (jnp.dot(pooled, fcw_ref[...], preferred_element_type=jnp.float32)
                + fcb_ref[...])


def _net(h0, qw, kw, vw, ow, qb, kb, vb, ob, g1, b1, w1, fb1, w2, fb2, g2, b2,
         fc_w, fc_b):
    mat = pl.BlockSpec((1, D, D), lambda b, i: (i, 0, 0))
    row = pl.BlockSpec((1, 1, D), lambda b, i: (i, 0, 0))
    return pl.pallas_call(
        _net_body,
        grid=(B, NL),
        in_specs=[
            pl.BlockSpec((1, L, D), lambda b, i: (b, 0, 0)),
            mat, mat, mat, mat,
            row, row, row, row, row, row,
            pl.BlockSpec((1, D, FF), lambda b, i: (i, 0, 0)),
            pl.BlockSpec((1, 1, FF), lambda b, i: (i, 0, 0)),
            pl.BlockSpec((1, FF, D), lambda b, i: (i, 0, 0)),
            row, row, row,
            pl.BlockSpec((D, HOR), lambda b, i: (0, 0)),
            pl.BlockSpec((1, HOR), lambda b, i: (0, 0)),
        ],
        out_specs=pl.BlockSpec((1, 1, HOR), lambda b, i: (b, 0, 0)),
        out_shape=jax.ShapeDtypeStruct((B, 1, HOR), jnp.float32),
        scratch_shapes=[pltpu.VMEM((L, D), jnp.float32)],
    )(h0, qw, kw, vw, ow, qb, kb, vb, ob, g1, b1, w1, fb1, w2, fb2, g2, b2,
      fc_w, fc_b.reshape(1, HOR))


def kernel(x, emb_w, emb_b, q_w, q_b, k_w, k_b, v_w, v_b, o_w, o_b,
           ff1_w, ff1_b, ff2_w, ff2_b, n1_g, n1_b, n2_g, n2_b, fc_w, fc_b):
    h0 = _embed(x, emb_w, emb_b)
    out = _net(h0,
               q_w.astype(jnp.bfloat16), k_w.astype(jnp.bfloat16),
               v_w.astype(jnp.bfloat16), o_w.astype(jnp.bfloat16),
               q_b.reshape(NL, 1, D), k_b.reshape(NL, 1, D),
               v_b.reshape(NL, 1, D), o_b.reshape(NL, 1, D),
               n1_g.reshape(NL, 1, D), n1_b.reshape(NL, 1, D),
               ff1_w, ff1_b.reshape(NL, 1, FF),
               ff2_w, ff2_b.reshape(NL, 1, D),
               n2_g.reshape(NL, 1, D), n2_b.reshape(NL, 1, D),
               fc_w, fc_b)
    return out.reshape(B, HOR)


# speedup vs baseline: 1.0627x; 1.0627x over previous
import math

import numpy as np
import jax
import jax.numpy as jnp
from jax.experimental import pallas as pl
from jax.experimental.pallas import tpu as pltpu

B = 2
P = 2048
D = 1024
H = 16
DK = 64
FF = 32
HOR = 24
NL = 2
L = D
U = 6
SCALE = float(DK ** 0.5)


def _pe_np():
    pe = np.zeros((L, D), np.float32)
    pos = np.arange(L, dtype=np.float32)[:, None]
    div = np.exp(np.arange(0, D, 2, dtype=np.float32) * (-math.log(10000.0) / D))
    pe[:, 0::2] = np.sin(pos * div)
    pe[:, 1::2] = np.cos(pos * div)
    return pe


_PE = _pe_np()

_RB = 256


def _embed_body(x_ref, w_ref, b_ref, pe_ref, o_ref):
    xb = x_ref[0]
    wb = w_ref[...]
    acc = jax.lax.dot_general(wb, xb, (((0,), (0,)), ((), ())),
                              preferred_element_type=jnp.float32)
    o_ref[0] = acc + b_ref[...] + pe_ref[...]


def _embed(x, emb_w, emb_b):
    return pl.pallas_call(
        _embed_body,
        grid=(B, L // _RB),
        in_specs=[
            pl.BlockSpec((1, P, D), lambda b, j: (b, 0, 0)),
            pl.BlockSpec((P, _RB), lambda b, j: (0, j)),
            pl.BlockSpec((_RB, 1), lambda b, j: (j, 0)),
            pl.BlockSpec((_RB, D), lambda b, j: (j, 0)),
        ],
        out_specs=pl.BlockSpec((1, _RB, D), lambda b, j: (b, j, 0)),
        out_shape=jax.ShapeDtypeStruct((B, L, D), jnp.float32),
    )(x, emb_w, emb_b.reshape(D, 1), jnp.asarray(_PE))


def _net_body(h0_ref, qw_ref, kw_ref, vw_ref, ow_ref, qb_ref, kb_ref, vb_ref,
              ob_ref, g1_ref, b1_ref, w1_ref, fb1_ref, w2_ref, fb2_ref,
              g2_ref, b2_ref, fcw_ref, fcb_ref, o_ref, hs_ref):
    i = pl.program_id(1)

    @pl.when(i == 0)
    def _():
        hs_ref[...] = h0_ref[0]

    hb = hs_ref[...]
    hb16 = hb.astype(jnp.bfloat16)
    q_all = jnp.dot(hb16, qw_ref[0],
                    preferred_element_type=jnp.float32) + qb_ref[0]
    k_all = jnp.dot(hb16, kw_ref[0],
                    preferred_element_type=jnp.float32) + kb_ref[0]
    q16 = q_all.astype(jnp.bfloat16)
    k16 = k_all.astype(jnp.bfloat16)
    io_row = jax.lax.broadcasted_iota(jnp.int32, (1, L), 1)
    m_list = []
    for hh in range(H):
        sl = slice(hh * DK, (hh + 1) * DK)
        st = jax.lax.dot_general(k16[:, sl], q16[:, sl],
                                 (((1,), (1,)), ((), ())),
                                 preferred_element_type=jnp.float32)
        m_list.append(jnp.max(st, axis=0, keepdims=True))
    m2 = jnp.concatenate(m_list, axis=0)
    idx_cols = []
    for _ in range(U):
        vmax = jnp.max(m2, axis=1, keepdims=True)
        idx2 = jnp.min(jnp.where(m2 == vmax, io_row, jnp.int32(L)),
                       axis=1, keepdims=True)
        idx_cols.append(idx2)
        m2 = jnp.where(io_row == idx2, jnp.float32(-jnp.inf), m2)
    pad = jnp.full((H, 1), jnp.int32(L), jnp.int32)
    idx_mat = jnp.concatenate(idx_cols + [pad, pad], axis=1)
    e_all = (idx_mat[:, :, None] == io_row.reshape(1, 1, L)
             ).astype(jnp.float32).reshape(H * 8, L)
    rg = jax.lax.broadcasted_iota(jnp.int32, (H * 8, 1), 0) // 8
    maskc = (rg == io_row // DK).astype(jnp.float32)
    q_sel = jnp.dot(e_all, q_all, preferred_element_type=jnp.float32) * maskc
    a = jax.lax.dot_general(q_sel, k_all, (((1,), (1,)), ((), ())),
                            preferred_element_type=jnp.float32) / SCALE
    a = a - jnp.max(a, axis=1, keepdims=True)
    w = jnp.exp(a)
    w = w / jnp.sum(w, axis=1, keepdims=True)
    wh = jnp.dot(w.astype(jnp.bfloat16), hb16,
                 preferred_element_type=jnp.float32)
    ctx = (jnp.dot(wh, vw_ref[0], preferred_element_type=jnp.float32)
           + vb_ref[0]) * maskc
    cf = jnp.dot(ctx, ow_ref[0], preferred_element_type=jnp.float32)
    delta = jax.lax.dot_general(e_all, cf, (((0,), (0,)), ((), ())),
                                preferred_element_type=jnp.float32)
    y = hb + delta + ob_ref[0]
    mu = jnp.mean(y, axis=1, keepdims=True)
    var = jnp.mean(y * y, axis=1, keepdims=True) - mu * mu
    hn = (y - mu) / jnp.sqrt(var + 1e-5) * g1_ref[0] + b1_ref[0]
    f = jnp.maximum(
        jnp.dot(hn, w1_ref[0], preferred_element_type=jnp.float32) + fb1_ref[0],
        0.0)
    f = jnp.dot(f, w2_ref[0], preferred_element_type=jnp.float32) + fb2_ref[0]
    z = hn + f
    mu2 = jnp.mean(z, axis=1, keepdims=True)
    var2 = jnp.mean(z * z, axis=1, keepdims=True) - mu2 * mu2
    zf = (z - mu2) / jnp.sqrt(var2 + 1e-5) * g2_ref[0] + b2_ref[0]
    hs_ref[...] = zf
    pooled = jnp.mean(zf, axis=0, keepdims=True)
    o_ref[0] = (jnp.dot(pooled, fcw_ref[...], preferred_element_type=jnp.float32)
                + fcb_ref[...])


def _net(h0, qw, kw, vw, ow, qb, kb, vb, ob, g1, b1, w1, fb1, w2, fb2, g2, b2,
         fc_w, fc_b):
    mat = pl.BlockSpec((1, D, D), lambda b, i: (i, 0, 0))
    row = pl.BlockSpec((1, 1, D), lambda b, i: (i, 0, 0))
    return pl.pallas_call(
        _net_body,
        grid=(B, NL),
        in_specs=[
            pl.BlockSpec((1, L, D), lambda b, i: (b, 0, 0)),
            mat, mat, mat, mat,
            row, row, row, row, row, row,
            pl.BlockSpec((1, D, FF), lambda b, i: (i, 0, 0)),
            pl.BlockSpec((1, 1, FF), lambda b, i: (i, 0, 0)),
            pl.BlockSpec((1, FF, D), lambda b, i: (i, 0, 0)),
            row, row, row,
            pl.BlockSpec((D, HOR), lambda b, i: (0, 0)),
            pl.BlockSpec((1, HOR), lambda b, i: (0, 0)),
        ],
        out_specs=pl.BlockSpec((1, 1, HOR), lambda b, i: (b, 0, 0)),
        out_shape=jax.ShapeDtypeStruct((B, 1, HOR), jnp.float32),
        scratch_shapes=[pltpu.VMEM((L, D), jnp.float32)],
    )(h0, qw, kw, vw, ow, qb, kb, vb, ob, g1, b1, w1, fb1, w2, fb2, g2, b2,
      fc_w, fc_b.reshape(1, HOR))


def kernel(x, emb_w, emb_b, q_w, q_b, k_w, k_b, v_w, v_b, o_w, o_b,
           ff1_w, ff1_b, ff2_w, ff2_b, n1_g, n1_b, n2_g, n2_b, fc_w, fc_b):
    h0 = _embed(x, emb_w, emb_b)
    out = _net(h0,
               q_w.astype(jnp.bfloat16), k_w.astype(jnp.bfloat16),
               v_w, o_w,
               q_b.reshape(NL, 1, D), k_b.reshape(NL, 1, D),
               v_b.reshape(NL, 1, D), o_b.reshape(NL, 1, D),
               n1_g.reshape(NL, 1, D), n1_b.reshape(NL, 1, D),
               ff1_w, ff1_b.reshape(NL, 1, FF),
               ff2_w, ff2_b.reshape(NL, 1, D),
               n2_g.reshape(NL, 1, D), n2_b.reshape(NL, 1, D),
               fc_w, fc_b)
    return out.reshape(B, HOR)


# speedup vs baseline: 1.1313x; 1.0646x over previous
import math

import numpy as np
import jax
import jax.numpy as jnp
from jax.experimental import pallas as pl
from jax.experimental.pallas import tpu as pltpu

B = 2
P = 2048
D = 1024
H = 16
DK = 64
FF = 32
HOR = 24
NL = 2
L = D
U = 6
SCALE = float(DK ** 0.5)


def _pe_np():
    pe = np.zeros((L, D), np.float32)
    pos = np.arange(L, dtype=np.float32)[:, None]
    div = np.exp(np.arange(0, D, 2, dtype=np.float32) * (-math.log(10000.0) / D))
    pe[:, 0::2] = np.sin(pos * div)
    pe[:, 1::2] = np.cos(pos * div)
    return pe


_PE = _pe_np()

_RB = 256


def _embed_body(x_ref, w_ref, b_ref, pe_ref, o_ref):
    xb = x_ref[0]
    wb = w_ref[...]
    acc = jax.lax.dot_general(wb, xb, (((0,), (0,)), ((), ())),
                              preferred_element_type=jnp.float32)
    o_ref[0] = acc + b_ref[...] + pe_ref[...]


def _embed(x, emb_w, emb_b):
    return pl.pallas_call(
        _embed_body,
        grid=(B, L // _RB),
        in_specs=[
            pl.BlockSpec((1, P, D), lambda b, j: (b, 0, 0)),
            pl.BlockSpec((P, _RB), lambda b, j: (0, j)),
            pl.BlockSpec((_RB, 1), lambda b, j: (j, 0)),
            pl.BlockSpec((_RB, D), lambda b, j: (j, 0)),
        ],
        out_specs=pl.BlockSpec((1, _RB, D), lambda b, j: (b, j, 0)),
        out_shape=jax.ShapeDtypeStruct((B, L, D), jnp.float32),
    )(x, emb_w, emb_b.reshape(D, 1), jnp.asarray(_PE))


def _net_body(h0_ref, qw_ref, kw_ref, vw_ref, ow_ref, qb_ref, kb_ref, vb_ref,
              ob_ref, g1_ref, b1_ref, w1_ref, fb1_ref, w2_ref, fb2_ref,
              g2_ref, b2_ref, fcw_ref, fcb_ref, o_ref, hs_ref):
    i = pl.program_id(1)

    @pl.when(i == 0)
    def _():
        hs_ref[...] = h0_ref[0]

    hb = hs_ref[...]
    hb16 = hb.astype(jnp.bfloat16)
    q_all = jnp.dot(hb, qw_ref[0],
                    preferred_element_type=jnp.float32) + qb_ref[0]
    k_all = jnp.dot(hb, kw_ref[0],
                    preferred_element_type=jnp.float32) + kb_ref[0]
    q16 = q_all.astype(jnp.bfloat16)
    k16 = k_all.astype(jnp.bfloat16)
    io_row = jax.lax.broadcasted_iota(jnp.int32, (1, L), 1)
    m_list = []
    for hh in range(H):
        sl = slice(hh * DK, (hh + 1) * DK)
        st = jax.lax.dot_general(k16[:, sl], q16[:, sl],
                                 (((1,), (1,)), ((), ())),
                                 preferred_element_type=jnp.float32)
        m_list.append(jnp.max(st, axis=0, keepdims=True))
    m2 = jnp.concatenate(m_list, axis=0)
    idx_cols = []
    for _ in range(U):
        vmax = jnp.max(m2, axis=1, keepdims=True)
        idx2 = jnp.min(jnp.where(m2 == vmax, io_row, jnp.int32(L)),
                       axis=1, keepdims=True)
        idx_cols.append(idx2)
        m2 = jnp.where(io_row == idx2, jnp.float32(-jnp.inf), m2)
    pad = jnp.full((H, 1), jnp.int32(L), jnp.int32)
    idx_mat = jnp.concatenate(idx_cols + [pad, pad], axis=1)
    e_all = (idx_mat[:, :, None] == io_row.reshape(1, 1, L)
             ).astype(jnp.float32).reshape(H * 8, L)
    rg = jax.lax.broadcasted_iota(jnp.int32, (H * 8, 1), 0) // 8
    maskc = (rg == io_row // DK).astype(jnp.float32)
    q_sel = jnp.dot(e_all, q_all, preferred_element_type=jnp.float32) * maskc
    a = jax.lax.dot_general(q_sel, k_all, (((1,), (1,)), ((), ())),
                            preferred_element_type=jnp.float32) / SCALE
    a = a - jnp.max(a, axis=1, keepdims=True)
    w = jnp.exp(a)
    w = w / jnp.sum(w, axis=1, keepdims=True)
    wh = jnp.dot(w.astype(jnp.bfloat16), hb16,
                 preferred_element_type=jnp.float32)
    ctx = (jnp.dot(wh, vw_ref[0], preferred_element_type=jnp.float32)
           + vb_ref[0]) * maskc
    cf = jnp.dot(ctx, ow_ref[0], preferred_element_type=jnp.float32)
    delta = jax.lax.dot_general(e_all, cf, (((0,), (0,)), ((), ())),
                                preferred_element_type=jnp.float32)
    y = hb + delta + ob_ref[0]
    mu = jnp.mean(y, axis=1, keepdims=True)
    var = jnp.mean(y * y, axis=1, keepdims=True) - mu * mu
    hn = (y - mu) / jnp.sqrt(var + 1e-5) * g1_ref[0] + b1_ref[0]
    f = jnp.maximum(
        jnp.dot(hn, w1_ref[0], preferred_element_type=jnp.float32) + fb1_ref[0],
        0.0)
    f = jnp.dot(f, w2_ref[0], preferred_element_type=jnp.float32) + fb2_ref[0]
    z = hn + f
    mu2 = jnp.mean(z, axis=1, keepdims=True)
    var2 = jnp.mean(z * z, axis=1, keepdims=True) - mu2 * mu2
    zf = (z - mu2) / jnp.sqrt(var2 + 1e-5) * g2_ref[0] + b2_ref[0]
    hs_ref[...] = zf
    pooled = jnp.mean(zf, axis=0, keepdims=True)
    o_ref[0] = (jnp.dot(pooled, fcw_ref[...], preferred_element_type=jnp.float32)
                + fcb_ref[...])


def _net(h0, qw, kw, vw, ow, qb, kb, vb, ob, g1, b1, w1, fb1, w2, fb2, g2, b2,
         fc_w, fc_b):
    mat = pl.BlockSpec((1, D, D), lambda b, i: (i, 0, 0))
    row = pl.BlockSpec((1, 1, D), lambda b, i: (i, 0, 0))
    return pl.pallas_call(
        _net_body,
        grid=(B, NL),
        in_specs=[
            pl.BlockSpec((1, L, D), lambda b, i: (b, 0, 0)),
            mat, mat, mat, mat,
            row, row, row, row, row, row,
            pl.BlockSpec((1, D, FF), lambda b, i: (i, 0, 0)),
            pl.BlockSpec((1, 1, FF), lambda b, i: (i, 0, 0)),
            pl.BlockSpec((1, FF, D), lambda b, i: (i, 0, 0)),
            row, row, row,
            pl.BlockSpec((D, HOR), lambda b, i: (0, 0)),
            pl.BlockSpec((1, HOR), lambda b, i: (0, 0)),
        ],
        out_specs=pl.BlockSpec((1, 1, HOR), lambda b, i: (b, 0, 0)),
        out_shape=jax.ShapeDtypeStruct((B, 1, HOR), jnp.float32),
        scratch_shapes=[pltpu.VMEM((L, D), jnp.float32)],
    )(h0, qw, kw, vw, ow, qb, kb, vb, ob, g1, b1, w1, fb1, w2, fb2, g2, b2,
      fc_w, fc_b.reshape(1, HOR))


def kernel(x, emb_w, emb_b, q_w, q_b, k_w, k_b, v_w, v_b, o_w, o_b,
           ff1_w, ff1_b, ff2_w, ff2_b, n1_g, n1_b, n2_g, n2_b, fc_w, fc_b):
    h0 = _embed(x, emb_w, emb_b)
    out = _net(h0,
               q_w, k_w, v_w, o_w,
               q_b.reshape(NL, 1, D), k_b.reshape(NL, 1, D),
               v_b.reshape(NL, 1, D), o_b.reshape(NL, 1, D),
               n1_g.reshape(NL, 1, D), n1_b.reshape(NL, 1, D),
               ff1_w, ff1_b.reshape(NL, 1, FF),
               ff2_w, ff2_b.reshape(NL, 1, D),
               n2_g.reshape(NL, 1, D), n2_b.reshape(NL, 1, D),
               fc_w, fc_b)
    return out.reshape(B, HOR)
